# baseline (speedup 1.0000x reference)
import functools

import jax
import jax.numpy as jnp
from jax import lax
from jax.experimental import pallas as pl
from jax.experimental.pallas import tpu as pltpu
from jax.experimental.pallas import tpu_sc as plsc

N_TOK = 2048
C = 768
H = 3072
E = 8
TA = 256
NT_A = N_TOK // TA
TR = 512
G_R = 11
NS_R = G_R * TR
T_SH = 512

SC_CORES = 2
SC_SUBCORES = 16
SC_WORKERS = SC_CORES * SC_SUBCORES


def _router_body(x_ref, gW_ref, gv_ref, slot_ref, te_ref,
                 ltri, running, eid_s, rank_s):
    g = pl.program_id(0)

    @pl.when(g == 0)
    def _():
        r = jax.lax.broadcasted_iota(jnp.int32, (TA, TA), 0)
        c = jax.lax.broadcasted_iota(jnp.int32, (TA, TA), 1)
        ltri[...] = (c < r).astype(jnp.bfloat16)
        running[...] = jnp.zeros((1, E), jnp.float32)

    @pl.when(g < NT_A)
    def _():
        xr = x_ref[...]
        logits = jax.lax.dot_general(
            xr, gW_ref[...], (((1,), (1,)), ((), ())),
            preferred_element_type=jnp.float32) + gv_ref[...]
        m = jnp.max(logits, axis=-1, keepdims=True)
        iota = jax.lax.broadcasted_iota(jnp.int32, logits.shape, 1)
        eid = jnp.min(jnp.where(logits == m, iota, E), axis=-1)
        onehot = (eid[:, None] == jax.lax.broadcasted_iota(
            jnp.int32, (TA, E), 1)).astype(jnp.float32)
        rank_t = jax.lax.dot_general(
            ltri[...], onehot.astype(jnp.bfloat16),
            (((1,), (0,)), ((), ())), preferred_element_type=jnp.float32)
        rank = (jnp.sum(rank_t * onehot, axis=1)
                + jnp.sum(running[...] * onehot, axis=1))
        eid_s[pl.ds(g * TA, TA), :] = eid[:, None]
        rank_s[pl.ds(g * TA, TA), :] = rank[:, None]
        running[...] += jnp.sum(onehot, axis=0, keepdims=True)

    @pl.when(g == NT_A)
    def _():
        counts = running[...]
        padded = jnp.floor((counts + (TR - 1)) / TR) * TR
        i8r = jax.lax.broadcasted_iota(jnp.int32, (E, E), 0)
        i8c = jax.lax.broadcasted_iota(jnp.int32, (E, E), 1)
        utri = (i8r <= i8c).astype(jnp.bfloat16)
        ends = jax.lax.dot_general(
            padded.astype(jnp.bfloat16), utri,
            (((1,), (0,)), ((), ())), preferred_element_type=jnp.float32)
        starts = ends - padded
        eid_all = eid_s[...]
        onehot_all = (eid_all == jax.lax.broadcasted_iota(
            jnp.int32, (N_TOK, E), 1)).astype(jnp.float32)
        slot = rank_s[..., 0] + jnp.sum(onehot_all * starts, axis=1)
        slot_ref[...] = slot.astype(jnp.int32)[:, None]
        tstart = (jax.lax.broadcasted_iota(jnp.int32, (G_R, 1), 0)
                  * TR).astype(jnp.float32)
        te = jnp.sum((ends <= tstart).astype(jnp.int32),
                     axis=1, keepdims=True)
        te_ref[...] = jnp.clip(te, 0, E - 1)


def _shared_body(x_ref, W1_ref, b1_ref, W2_ref, b2_ref, out_ref):
    xb = x_ref[...].astype(jnp.bfloat16)
    h = jax.lax.dot_general(
        xb, W1_ref[...].astype(jnp.bfloat16),
        (((1,), (1,)), ((), ())), preferred_element_type=jnp.float32)
    h = jnp.maximum(h + b1_ref[0], 0.0)
    y = jax.lax.dot_general(
        h.astype(jnp.bfloat16), W2_ref[...].astype(jnp.bfloat16),
        (((1,), (1,)), ((), ())), preferred_element_type=jnp.float32)
    out_ref[...] = y + b2_ref[0]


def _grouped_body(te_ref, xs_ref, W1a_ref, W1c_ref, b1_ref,
                  W2a_ref, W2c_ref, b2_ref, ys_ref, W1b, W2b):
    g = pl.program_id(0)
    prev = te_ref[jnp.maximum(g - 1, 0)]
    new_expert = (g == 0) | (te_ref[g] != prev)
    H2 = H // 2

    @pl.when(new_expert)
    def _():
        W1b[0:H2, :] = W1a_ref[0, 0].astype(jnp.bfloat16)
        W1b[H2:H, :] = W1c_ref[0, 0].astype(jnp.bfloat16)
        C2 = C // 2
        W2b[0:C2, :] = W2a_ref[0, 0].astype(jnp.bfloat16)
        W2b[C2:C, :] = W2c_ref[0, 0].astype(jnp.bfloat16)

    h = jax.lax.dot_general(
        xs_ref[...].astype(jnp.bfloat16), W1b[...],
        (((1,), (1,)), ((), ())),
        preferred_element_type=jnp.float32)
    h = jnp.maximum(h + b1_ref[0, 0], 0.0)
    y = jax.lax.dot_general(
        h.astype(jnp.bfloat16), W2b[...], (((1,), (1,)), ((), ())),
        preferred_element_type=jnp.float32)
    ys_ref[...] = y + b2_ref[0, 0]


def _combine_body(a1_ref, a2_ref, b_ref, out_ref):
    t = pl.program_id(0)

    @pl.when(t == 0)
    def _():
        out_ref[...] = a1_ref[...] + b_ref[...]

    @pl.when(t > 0)
    def _():
        out_ref[...] = a2_ref[...] + b_ref[...]


def _sc_scatter(rows, idx, n_out):
    n = rows.shape[0]
    rows_per_w = n // SC_WORKERS
    mesh = plsc.VectorSubcoreMesh(core_axis_name="c", subcore_axis_name="s")

    @functools.partial(
        pl.kernel, mesh=mesh,
        out_type=jax.ShapeDtypeStruct((n_out, C), jnp.float32),
        scratch_types=[
            pltpu.VMEM((rows_per_w,), jnp.int32),
            pltpu.VMEM((rows_per_w, C), jnp.float32),
        ],
    )
    def k(rows_hbm, idx_hbm, out_hbm, idx_v, rows_v):
        wid = lax.axis_index("s") * SC_CORES + lax.axis_index("c")
        base = wid * rows_per_w
        pltpu.sync_copy(idx_hbm.at[pl.ds(base, rows_per_w)], idx_v)
        pltpu.sync_copy(rows_hbm.at[pl.ds(base, rows_per_w)], rows_v)
        pltpu.sync_copy(rows_v, out_hbm.at[idx_v])

    return k(rows, idx)


def _sc_gather(table, idx, n_rows):
    rows_per_w = n_rows // SC_WORKERS
    mesh = plsc.VectorSubcoreMesh(core_axis_name="c", subcore_axis_name="s")

    @functools.partial(
        pl.kernel, mesh=mesh,
        out_type=jax.ShapeDtypeStruct((n_rows, C), jnp.float32),
        scratch_types=[
            pltpu.VMEM((rows_per_w,), jnp.int32),
            pltpu.VMEM((rows_per_w, C), jnp.float32),
            pltpu.SemaphoreType.DMA,
        ],
    )
    def k(table_hbm, idx_hbm, out_hbm, idx_v, rows_v, sem):
        wid = lax.axis_index("s") * SC_CORES + lax.axis_index("c")
        base = wid * rows_per_w
        pltpu.sync_copy(idx_hbm.at[pl.ds(base, rows_per_w)], idx_v)
        pltpu.async_copy(table_hbm.at[idx_v], rows_v, sem).wait()
        pltpu.sync_copy(rows_v, out_hbm.at[pl.ds(base, rows_per_w)])

    return k(table, idx)


def kernel(x, sW1, sb1, sW2, sb2, eW1, eb1, eW2, eb2, gW, gb, gate_bias):
    xs2 = x.reshape(N_TOK, C)
    gv = (gb + gate_bias).reshape(1, E)

    slot2, te2 = pl.pallas_call(
        _router_body,
        grid=(NT_A + 1,),
        in_specs=[
            pl.BlockSpec((TA, C), lambda g: (jnp.minimum(g, NT_A - 1), 0)),
            pl.BlockSpec((E, C), lambda g: (0, 0)),
            pl.BlockSpec((1, E), lambda g: (0, 0)),
        ],
        out_specs=[
            pl.BlockSpec((N_TOK, 1), lambda g: (0, 0)),
            pl.BlockSpec((G_R, 1), lambda g: (0, 0)),
        ],
        out_shape=[
            jax.ShapeDtypeStruct((N_TOK, 1), jnp.int32),
            jax.ShapeDtypeStruct((G_R, 1), jnp.int32),
        ],
        scratch_shapes=[
            pltpu.VMEM((TA, TA), jnp.bfloat16),
            pltpu.VMEM((1, E), jnp.float32),
            pltpu.VMEM((N_TOK, 1), jnp.int32),
            pltpu.VMEM((N_TOK, 1), jnp.float32),
        ],
    )(xs2, gW, gv)
    slot = slot2[:, 0]
    tile_eid = te2[:, 0]

    xs_sorted = _sc_scatter(xs2, slot, NS_R)

    sh1 = pl.pallas_call(
        _shared_body,
        grid=(1,),
        in_specs=[
            pl.BlockSpec((T_SH, C), lambda t: (0, 0)),
            pl.BlockSpec((H, C), lambda t: (0, 0)),
            pl.BlockSpec((1, H), lambda t: (0, 0)),
            pl.BlockSpec((C, H), lambda t: (0, 0)),
            pl.BlockSpec((1, C), lambda t: (0, 0)),
        ],
        out_specs=pl.BlockSpec((T_SH, C), lambda t: (0, 0)),
        out_shape=jax.ShapeDtypeStruct((T_SH, C), jnp.float32),
    )(xs2, sW1, sb1.reshape(1, H), sW2, sb2.reshape(1, C))
    xs_sorted, sh1 = lax.optimization_barrier((xs_sorted, sh1))

    ys = pl.pallas_call(
        _grouped_body,
        grid_spec=pltpu.PrefetchScalarGridSpec(
            num_scalar_prefetch=1,
            grid=(G_R,),
            in_specs=[
                pl.BlockSpec((TR, C), lambda g, te: (g, 0)),
                pl.BlockSpec((1, 1, H // 2, C), lambda g, te: (te[g], 0, 0, 0)),
                pl.BlockSpec((1, 1, H // 2, C), lambda g, te: (te[g], 1, 0, 0)),
                pl.BlockSpec((1, 1, H), lambda g, te: (te[g], 0, 0)),
                pl.BlockSpec((1, 1, C // 2, H), lambda g, te: (te[g], 0, 0, 0)),
                pl.BlockSpec((1, 1, C // 2, H), lambda g, te: (te[g], 1, 0, 0)),
                pl.BlockSpec((1, 1, C), lambda g, te: (te[g], 0, 0)),
            ],
            out_specs=pl.BlockSpec((TR, C), lambda g, te: (g, 0)),
            scratch_shapes=[
                pltpu.VMEM((H, C), jnp.bfloat16),
                pltpu.VMEM((C, H), jnp.bfloat16),
            ],
        ),
        out_shape=jax.ShapeDtypeStruct((NS_R, C), jnp.float32),
    )(tile_eid, xs_sorted, eW1.reshape(E, 2, H // 2, C),
      eW1.reshape(E, 2, H // 2, C), eb1.reshape(E, 1, H),
      eW2.reshape(E, 2, C // 2, H), eW2.reshape(E, 2, C // 2, H),
      eb2.reshape(E, 1, C))

    xs2b, ys = lax.optimization_barrier((xs2, ys))
    yu = _sc_gather(ys, slot, N_TOK)

    sh2 = pl.pallas_call(
        _shared_body,
        grid=(N_TOK // T_SH - 1,),
        in_specs=[
            pl.BlockSpec((T_SH, C), lambda t: (t + 1, 0)),
            pl.BlockSpec((H, C), lambda t: (0, 0)),
            pl.BlockSpec((1, H), lambda t: (0, 0)),
            pl.BlockSpec((C, H), lambda t: (0, 0)),
            pl.BlockSpec((1, C), lambda t: (0, 0)),
        ],
        out_specs=pl.BlockSpec((T_SH, C), lambda t: (t, 0)),
        out_shape=jax.ShapeDtypeStruct((N_TOK - T_SH, C), jnp.float32),
    )(xs2b, sW1, sb1.reshape(1, H), sW2, sb2.reshape(1, C))

    out = pl.pallas_call(
        _combine_body,
        grid=(N_TOK // T_SH,),
        in_specs=[
            pl.BlockSpec((T_SH, C), lambda t: (0, 0)),
            pl.BlockSpec((T_SH, C), lambda t: (jnp.maximum(t - 1, 0), 0)),
            pl.BlockSpec((T_SH, C), lambda t: (t, 0)),
        ],
        out_specs=pl.BlockSpec((T_SH, C), lambda t: (t, 0)),
        out_shape=jax.ShapeDtypeStruct((N_TOK, C), jnp.float32),
    )(sh1, sh2, yu)

    return out.reshape(x.shape)


# speedup vs baseline: 1.0407x; 1.0407x over previous
import functools

import jax
import jax.numpy as jnp
from jax import lax
from jax.experimental import pallas as pl
from jax.experimental.pallas import tpu as pltpu
from jax.experimental.pallas import tpu_sc as plsc

N_TOK = 2048
C = 768
H = 3072
E = 8
TA = 256
NT_A = N_TOK // TA
TR = 512
G_R = 11
NS_R = G_R * TR
T_SH = 512

SC_CORES = 2
SC_SUBCORES = 16
SC_WORKERS = SC_CORES * SC_SUBCORES


def _router_body(x_ref, gW_ref, gv_ref, slot_ref, te_ref,
                 ltri, running, eid_s, rank_s):
    g = pl.program_id(0)

    @pl.when(g == 0)
    def _():
        r = jax.lax.broadcasted_iota(jnp.int32, (TA, TA), 0)
        c = jax.lax.broadcasted_iota(jnp.int32, (TA, TA), 1)
        ltri[...] = (c < r).astype(jnp.bfloat16)
        running[...] = jnp.zeros((1, E), jnp.float32)

    @pl.when(g < NT_A)
    def _():
        xr = x_ref[...]
        logits = jax.lax.dot_general(
            xr, gW_ref[...], (((1,), (1,)), ((), ())),
            preferred_element_type=jnp.float32) + gv_ref[...]
        m = jnp.max(logits, axis=-1, keepdims=True)
        iota = jax.lax.broadcasted_iota(jnp.int32, logits.shape, 1)
        eid = jnp.min(jnp.where(logits == m, iota, E), axis=-1)
        onehot = (eid[:, None] == jax.lax.broadcasted_iota(
            jnp.int32, (TA, E), 1)).astype(jnp.float32)
        rank_t = jax.lax.dot_general(
            ltri[...], onehot.astype(jnp.bfloat16),
            (((1,), (0,)), ((), ())), preferred_element_type=jnp.float32)
        rank = (jnp.sum(rank_t * onehot, axis=1)
                + jnp.sum(running[...] * onehot, axis=1))
        eid_s[pl.ds(g * TA, TA), :] = eid[:, None]
        rank_s[pl.ds(g * TA, TA), :] = rank[:, None]
        running[...] += jnp.sum(onehot, axis=0, keepdims=True)

    @pl.when(g == NT_A)
    def _():
        counts = running[...]
        padded = jnp.floor((counts + (TR - 1)) / TR) * TR
        i8r = jax.lax.broadcasted_iota(jnp.int32, (E, E), 0)
        i8c = jax.lax.broadcasted_iota(jnp.int32, (E, E), 1)
        utri = (i8r <= i8c).astype(jnp.bfloat16)
        ends = jax.lax.dot_general(
            padded.astype(jnp.bfloat16), utri,
            (((1,), (0,)), ((), ())), preferred_element_type=jnp.float32)
        starts = ends - padded
        eid_all = eid_s[...]
        onehot_all = (eid_all == jax.lax.broadcasted_iota(
            jnp.int32, (N_TOK, E), 1)).astype(jnp.float32)
        slot = rank_s[..., 0] + jnp.sum(onehot_all * starts, axis=1)
        slot_ref[...] = slot.astype(jnp.int32)[:, None]
        tstart = (jax.lax.broadcasted_iota(jnp.int32, (G_R, 1), 0)
                  * TR).astype(jnp.float32)
        te = jnp.sum((ends <= tstart).astype(jnp.int32),
                     axis=1, keepdims=True)
        te_ref[...] = jnp.clip(te, 0, E - 1)


def _shared_body(x_ref, W1_ref, b1_ref, W2_ref, b2_ref, out_ref):
    xb = x_ref[...].astype(jnp.bfloat16)
    h = jax.lax.dot_general(
        xb, W1_ref[...].astype(jnp.bfloat16),
        (((1,), (1,)), ((), ())), preferred_element_type=jnp.float32)
    h = jnp.maximum(h + b1_ref[0], 0.0)
    y = jax.lax.dot_general(
        h.astype(jnp.bfloat16), W2_ref[...].astype(jnp.bfloat16),
        (((1,), (1,)), ((), ())), preferred_element_type=jnp.float32)
    out_ref[...] = y + b2_ref[0]


def _grouped_body(te_ref, xs_ref, W1a_ref, W1c_ref, b1_ref,
                  W2a_ref, W2c_ref, b2_ref, ys_ref, W1b, W2b):
    g = pl.program_id(0)
    prev = te_ref[jnp.maximum(g - 1, 0)]
    new_expert = (g == 0) | (te_ref[g] != prev)
    H2 = H // 2

    @pl.when(new_expert)
    def _():
        W1b[0:H2, :] = W1a_ref[0, 0].astype(jnp.bfloat16)
        W1b[H2:H, :] = W1c_ref[0, 0].astype(jnp.bfloat16)
        C2 = C // 2
        W2b[0:C2, :] = W2a_ref[0, 0].astype(jnp.bfloat16)
        W2b[C2:C, :] = W2c_ref[0, 0].astype(jnp.bfloat16)

    h = jax.lax.dot_general(
        xs_ref[...].astype(jnp.bfloat16), W1b[...],
        (((1,), (1,)), ((), ())),
        preferred_element_type=jnp.float32)
    h = jnp.maximum(h + b1_ref[0, 0], 0.0)
    y = jax.lax.dot_general(
        h.astype(jnp.bfloat16), W2b[...], (((1,), (1,)), ((), ())),
        preferred_element_type=jnp.float32)
    ys_ref[...] = y + b2_ref[0, 0]


def _combine_body(a_ref, b_ref, out_ref):
    out_ref[...] = a_ref[...] + b_ref[...]


def _sc_scatter(rows, idx, n_out):
    n = rows.shape[0]
    rows_per_w = n // SC_WORKERS
    mesh = plsc.VectorSubcoreMesh(core_axis_name="c", subcore_axis_name="s")

    @functools.partial(
        pl.kernel, mesh=mesh,
        out_type=jax.ShapeDtypeStruct((n_out, C), jnp.float32),
        scratch_types=[
            pltpu.VMEM((rows_per_w,), jnp.int32),
            pltpu.VMEM((rows_per_w, C), jnp.float32),
        ],
    )
    def k(rows_hbm, idx_hbm, out_hbm, idx_v, rows_v):
        wid = lax.axis_index("s") * SC_CORES + lax.axis_index("c")
        base = wid * rows_per_w
        pltpu.sync_copy(idx_hbm.at[pl.ds(base, rows_per_w)], idx_v)
        pltpu.sync_copy(rows_hbm.at[pl.ds(base, rows_per_w)], rows_v)
        pltpu.sync_copy(rows_v, out_hbm.at[idx_v])

    return k(rows, idx)


def _sc_gather(table, idx, n_rows):
    rows_per_w = n_rows // SC_WORKERS
    mesh = plsc.VectorSubcoreMesh(core_axis_name="c", subcore_axis_name="s")

    @functools.partial(
        pl.kernel, mesh=mesh,
        out_type=jax.ShapeDtypeStruct((n_rows, C), jnp.float32),
        scratch_types=[
            pltpu.VMEM((rows_per_w,), jnp.int32),
            pltpu.VMEM((rows_per_w, C), jnp.float32),
            pltpu.SemaphoreType.DMA,
        ],
    )
    def k(table_hbm, idx_hbm, out_hbm, idx_v, rows_v, sem):
        wid = lax.axis_index("s") * SC_CORES + lax.axis_index("c")
        base = wid * rows_per_w
        pltpu.sync_copy(idx_hbm.at[pl.ds(base, rows_per_w)], idx_v)
        pltpu.async_copy(table_hbm.at[idx_v], rows_v, sem).wait()
        pltpu.sync_copy(rows_v, out_hbm.at[pl.ds(base, rows_per_w)])

    return k(table, idx)


def kernel(x, sW1, sb1, sW2, sb2, eW1, eb1, eW2, eb2, gW, gb, gate_bias):
    xs2 = x.reshape(N_TOK, C)
    gv = (gb + gate_bias).reshape(1, E)

    slot2, te2 = pl.pallas_call(
        _router_body,
        grid=(NT_A + 1,),
        in_specs=[
            pl.BlockSpec((TA, C), lambda g: (jnp.minimum(g, NT_A - 1), 0)),
            pl.BlockSpec((E, C), lambda g: (0, 0)),
            pl.BlockSpec((1, E), lambda g: (0, 0)),
        ],
        out_specs=[
            pl.BlockSpec((N_TOK, 1), lambda g: (0, 0)),
            pl.BlockSpec((G_R, 1), lambda g: (0, 0)),
        ],
        out_shape=[
            jax.ShapeDtypeStruct((N_TOK, 1), jnp.int32),
            jax.ShapeDtypeStruct((G_R, 1), jnp.int32),
        ],
        scratch_shapes=[
            pltpu.VMEM((TA, TA), jnp.bfloat16),
            pltpu.VMEM((1, E), jnp.float32),
            pltpu.VMEM((N_TOK, 1), jnp.int32),
            pltpu.VMEM((N_TOK, 1), jnp.float32),
        ],
    )(xs2, gW, gv)
    slot = slot2[:, 0]
    tile_eid = te2[:, 0]

    xs_sorted = _sc_scatter(xs2, slot, NS_R)

    ys = pl.pallas_call(
        _grouped_body,
        grid_spec=pltpu.PrefetchScalarGridSpec(
            num_scalar_prefetch=1,
            grid=(G_R,),
            in_specs=[
                pl.BlockSpec((TR, C), lambda g, te: (g, 0)),
                pl.BlockSpec((1, 1, H // 2, C), lambda g, te: (te[g], 0, 0, 0)),
                pl.BlockSpec((1, 1, H // 2, C), lambda g, te: (te[g], 1, 0, 0)),
                pl.BlockSpec((1, 1, H), lambda g, te: (te[g], 0, 0)),
                pl.BlockSpec((1, 1, C // 2, H), lambda g, te: (te[g], 0, 0, 0)),
                pl.BlockSpec((1, 1, C // 2, H), lambda g, te: (te[g], 1, 0, 0)),
                pl.BlockSpec((1, 1, C), lambda g, te: (te[g], 0, 0)),
            ],
            out_specs=pl.BlockSpec((TR, C), lambda g, te: (g, 0)),
            scratch_shapes=[
                pltpu.VMEM((H, C), jnp.bfloat16),
                pltpu.VMEM((C, H), jnp.bfloat16),
            ],
        ),
        out_shape=jax.ShapeDtypeStruct((NS_R, C), jnp.float32),
    )(tile_eid, xs_sorted, eW1.reshape(E, 2, H // 2, C),
      eW1.reshape(E, 2, H // 2, C), eb1.reshape(E, 1, H),
      eW2.reshape(E, 2, C // 2, H), eW2.reshape(E, 2, C // 2, H),
      eb2.reshape(E, 1, C))

    yu = _sc_gather(ys, slot, N_TOK)

    shared_out = pl.pallas_call(
        _shared_body,
        grid=(N_TOK // T_SH,),
        in_specs=[
            pl.BlockSpec((T_SH, C), lambda t: (t, 0)),
            pl.BlockSpec((H, C), lambda t: (0, 0)),
            pl.BlockSpec((1, H), lambda t: (0, 0)),
            pl.BlockSpec((C, H), lambda t: (0, 0)),
            pl.BlockSpec((1, C), lambda t: (0, 0)),
        ],
        out_specs=pl.BlockSpec((T_SH, C), lambda t: (t, 0)),
        out_shape=jax.ShapeDtypeStruct((N_TOK, C), jnp.float32),
    )(xs2, sW1, sb1.reshape(1, H), sW2, sb2.reshape(1, C))

    out = pl.pallas_call(
        _combine_body,
        grid=(N_TOK // T_SH,),
        in_specs=[
            pl.BlockSpec((T_SH, C), lambda t: (t, 0)),
            pl.BlockSpec((T_SH, C), lambda t: (t, 0)),
        ],
        out_specs=pl.BlockSpec((T_SH, C), lambda t: (t, 0)),
        out_shape=jax.ShapeDtypeStruct((N_TOK, C), jnp.float32),
    )(shared_out, yu)

    return out.reshape(x.shape)
